# baseline (speedup 1.0000x reference)
import functools

import jax
import jax.numpy as jnp
from jax import lax
from jax.experimental import pallas as pl
from jax.experimental.pallas import tpu as pltpu
from jax.experimental.pallas import tpu_sc as plsc

_T = 16384
_D = 4096
_E = 64
_K = 8
_BT = 1024
_TH = _T // 2
_GRID_H = _TH // _BT
_GRID = 2 * _GRID_H

_NW = 32
_TPW = _TH // _NW
_CH = 128
_NCHUNK = _TPW // _CH


def _tc_body(w_ref, wt_ref, x_ref, b_ref, bt_ref,
             probs_ref, psum2_ref, idx_ref, prob_ref, psum1_ref, fsum1_ref,
             acc2_ref, ps1_ref, fs1_ref):
    step = pl.program_id(0)

    @pl.when(step == 0)
    def _init():
        acc2_ref[...] = jnp.zeros_like(acc2_ref)
        ps1_ref[...] = jnp.zeros_like(ps1_ref)
        fs1_ref[...] = jnp.zeros_like(fs1_ref)

    @pl.when(step < _GRID_H)
    def _half2_probs():
        logits = jnp.dot(x_ref[...], wt_ref[...],
                         preferred_element_type=jnp.float32)
        logits = logits + bt_ref[...]
        m = jnp.max(logits, axis=-1, keepdims=True)
        e = jnp.exp(logits - m)
        s = jnp.sum(e, axis=-1, keepdims=True)
        probs = e / s
        probs_ref[...] = probs
        acc2_ref[...] += jnp.sum(probs, axis=0, keepdims=True)

    @pl.when(step == _GRID_H - 1)
    def _fin2():
        psum2_ref[...] = acc2_ref[...]

    @pl.when(step >= _GRID_H)
    def _half1_fused():
        logits_t = lax.dot_general(
            w_ref[...], x_ref[...],
            dimension_numbers=(((1,), (1,)), ((), ())),
            preferred_element_type=jnp.float32)
        logits_t = logits_t + b_ref[...]

        row = lax.broadcasted_iota(jnp.int32, (_E, _BT), 0)
        neg_inf = jnp.float32(-jnp.inf)
        big = jnp.int32(_E)

        work = logits_t
        vals = []
        idxs = []
        for _ in range(_K):
            mx = jnp.max(work, axis=0, keepdims=True)
            hit0 = work == mx
            rsel = jnp.min(jnp.where(hit0, row, big), axis=0, keepdims=True)
            vals.append(mx)
            idxs.append(rsel)
            work = jnp.where(row == rsel, neg_inf, work)

        v8 = jnp.concatenate(vals, axis=0)
        i8 = jnp.concatenate(idxs, axis=0)

        m0 = vals[0]
        e_t = jnp.exp(logits_t - m0)
        zinv = jnp.float32(1.0) / jnp.sum(e_t, axis=0, keepdims=True)
        probs_t = e_t * zinv
        ps1_ref[...] += jnp.sum(probs_t, axis=1, keepdims=True)

        sel = (work == neg_inf).astype(jnp.float32)
        fs1_ref[...] += jnp.sum(sel, axis=1, keepdims=True)

        p8 = jnp.exp(v8 - m0) * zinv
        s8 = jnp.sum(p8, axis=0, keepdims=True)
        out_p = p8 / (s8 + jnp.float32(1e-9))

        prob_ref[...] = out_p.T
        idx_ref[...] = i8.T

    @pl.when(step == _GRID - 1)
    def _fin1():
        psum1_ref[...] = ps1_ref[...]
        fsum1_ref[...] = fs1_ref[...]


def _tc_main(x, W, b):
    return pl.pallas_call(
        _tc_body,
        grid=(_GRID,),
        in_specs=[
            pl.BlockSpec((_E, _D), lambda i: (0, 0)),
            pl.BlockSpec((_D, _E), lambda i: (0, 0)),
            pl.BlockSpec((_BT, _D), lambda i: ((i + _GRID_H) % _GRID, 0)),
            pl.BlockSpec((_E, 1), lambda i: (0, 0)),
            pl.BlockSpec((1, _E), lambda i: (0, 0)),
        ],
        out_specs=[
            pl.BlockSpec((_BT, _E), lambda i: (jnp.minimum(i, _GRID_H - 1), 0)),
            pl.BlockSpec((1, _E), lambda i: (0, 0)),
            pl.BlockSpec((_BT, _K), lambda i: (jnp.maximum(i - _GRID_H, 0), 0)),
            pl.BlockSpec((_BT, _K), lambda i: (jnp.maximum(i - _GRID_H, 0), 0)),
            pl.BlockSpec((_E, 1), lambda i: (0, 0)),
            pl.BlockSpec((_E, 1), lambda i: (0, 0)),
        ],
        out_shape=[
            jax.ShapeDtypeStruct((_TH, _E), jnp.float32),
            jax.ShapeDtypeStruct((1, _E), jnp.float32),
            jax.ShapeDtypeStruct((_TH, _K), jnp.int32),
            jax.ShapeDtypeStruct((_TH, _K), jnp.float32),
            jax.ShapeDtypeStruct((_E, 1), jnp.float32),
            jax.ShapeDtypeStruct((_E, 1), jnp.float32),
        ],
        scratch_shapes=[
            pltpu.VMEM((1, _E), jnp.float32),
            pltpu.VMEM((_E, 1), jnp.float32),
            pltpu.VMEM((_E, 1), jnp.float32),
        ],
        compiler_params=pltpu.CompilerParams(
            dimension_semantics=("arbitrary",),
        ),
    )(W, W.T, x, b.reshape(_E, 1), b.reshape(1, _E))


def _merge_top16(ka, va, kb, vb):
    kbr = lax.rev(kb, (0,))
    vbr = lax.rev(vb, (0,))
    ga = ka >= kbr
    km = jnp.where(ga, ka, kbr)
    vm = jnp.where(ga, va, vbr)
    return plsc.sort_key_val(km, vm, descending=True)


def _sc_router_body(probs_hbm, idxf_hbm, probf_hbm, fcnt_hbm,
                    pv_ref, stg_p, stg_i, cnt_ref, sem):
    c = lax.axis_index("c")
    s = lax.axis_index("s")
    wid = s * 2 + c
    base_tok = wid * _TPW

    lane = lax.broadcasted_iota(jnp.int32, (16,), 0)
    first8 = lane < 8
    ones16 = jnp.ones((16,), jnp.float32)
    zeros16 = jnp.zeros((16,), jnp.float32)
    for j in range(4):
        cnt_ref[pl.ds(16 * j, 16)] = zeros16

    def chunk_body(ci, carry):
        tok0 = base_tok + ci * _CH
        pltpu.sync_copy(probs_hbm.at[pl.ds(tok0, _CH)], pv_ref)

        @plsc.parallel_loop(0, _CH, unroll=8)
        def tok_body(t):
            p0 = pv_ref[t, pl.ds(0, 16)]
            p1 = pv_ref[t, pl.ds(16, 16)]
            p2 = pv_ref[t, pl.ds(32, 16)]
            p3 = pv_ref[t, pl.ds(48, 16)]
            k0, v0 = plsc.sort_key_val(p0, lane, descending=True)
            k1, v1 = plsc.sort_key_val(p1, lane + 16, descending=True)
            k2, v2 = plsc.sort_key_val(p2, lane + 32, descending=True)
            k3, v3 = plsc.sort_key_val(p3, lane + 48, descending=True)
            ka, va = _merge_top16(k0, v0, k1, v1)
            kb, vb = _merge_top16(k2, v2, k3, v3)
            kf, vf = _merge_top16(ka, va, kb, vb)

            s8 = jnp.sum(jnp.where(first8, kf, zeros16))
            outp = kf / (s8 + jnp.float32(1e-9))

            off = pl.multiple_of(t * _K, 8)
            plsc.store_compressed(stg_p.at[pl.ds(off, 16)], outp, mask=first8)
            plsc.store_compressed(stg_i.at[pl.ds(off, 16)], vf, mask=first8)

        def cnt_body(g, carry2):
            iv = stg_i[pl.ds(g * 16, 16)]
            plsc.addupdate_scatter(cnt_ref, [iv], ones16)
            return carry2

        lax.fori_loop(0, _CH * _K // 16, cnt_body, 0)
        flat0 = pl.multiple_of(tok0 * _K, 8)
        pltpu.sync_copy(stg_p.at[pl.ds(0, _CH * _K)],
                        probf_hbm.at[pl.ds(flat0, _CH * _K)])
        pltpu.sync_copy(stg_i.at[pl.ds(0, _CH * _K)],
                        idxf_hbm.at[pl.ds(flat0, _CH * _K)])
        return carry

    lax.fori_loop(0, _NCHUNK, chunk_body, 0)
    pltpu.sync_copy(cnt_ref, fcnt_hbm.at[wid])


_sc_router = functools.partial(
    pl.kernel,
    mesh=plsc.VectorSubcoreMesh(core_axis_name="c", subcore_axis_name="s"),
    compiler_params=pltpu.CompilerParams(needs_layout_passes=False),
    out_type=[
        jax.ShapeDtypeStruct((_TH * _K,), jnp.int32),
        jax.ShapeDtypeStruct((_TH * _K,), jnp.float32),
        jax.ShapeDtypeStruct((_NW, _E), jnp.float32),
    ],
    scratch_types=[
        pltpu.VMEM((_CH, _E), jnp.float32),
        pltpu.VMEM((_CH * _K + 8,), jnp.float32),
        pltpu.VMEM((_CH * _K + 8,), jnp.int32),
        pltpu.VMEM((_E,), jnp.float32),
        pltpu.SemaphoreType.DMA,
    ],
)(_sc_router_body)


def _aux_body(p1_ref, f1_ref, p2_ref, fc2_ref, aux_ref):
    f2 = jnp.sum(fc2_ref[...], axis=0, keepdims=True)
    nn = ((), ())
    a = lax.dot_general(p1_ref[...], f1_ref[...], (((0,), (0,)), nn),
                        preferred_element_type=jnp.float32)
    bb = lax.dot_general(p2_ref[...], f2, (((1,), (1,)), nn),
                         preferred_element_type=jnp.float32)
    cc = lax.dot_general(f2, p1_ref[...], (((1,), (0,)), nn),
                         preferred_element_type=jnp.float32)
    dd = lax.dot_general(p2_ref[...], f1_ref[...], (((1,), (0,)), nn),
                         preferred_element_type=jnp.float32)
    scale = jnp.float32(float(_E) / (float(_T) * float(_T) * float(_K)))
    aux_ref[...] = scale * (a + bb + cc + dd)


def _tc_aux(psum1, fsum1, psum2, fcnt2):
    return pl.pallas_call(
        _aux_body,
        out_shape=jax.ShapeDtypeStruct((1, 1), jnp.float32),
    )(psum1, fsum1, psum2, fcnt2)


def kernel(x, W, b):
    probs2, psum2, idx1, prob1, psum1, fsum1 = _tc_main(x, W, b)
    idx2, prob2, fcnt2 = _sc_router(probs2)
    aux = _tc_aux(psum1, fsum1, psum2, fcnt2)
    idx = jnp.concatenate([idx1, idx2.reshape(_TH, _K)], axis=0)
    prob = jnp.concatenate([prob1, prob2.reshape(_TH, _K)], axis=0)
    return (idx, prob, aux[0, 0])


# speedup vs baseline: 1.0175x; 1.0175x over previous
import functools

import jax
import jax.numpy as jnp
from jax import lax
from jax.experimental import pallas as pl
from jax.experimental.pallas import tpu as pltpu
from jax.experimental.pallas import tpu_sc as plsc

_T = 16384
_D = 4096
_E = 64
_K = 8
_BT = 1024
_TH = _T // 2
_GRID_H = _TH // _BT
_GRID = 2 * _GRID_H

_NW = 32
_TPW = _TH // _NW
_CH = 128
_NCHUNK = _TPW // _CH


def _tc_body(w_ref, wt_ref, x_ref, b_ref, bt_ref,
             probs_ref, psum2_ref, idx_ref, prob_ref, psum1_ref, fsum1_ref,
             acc2_ref, ps1_ref, fs1_ref):
    step = pl.program_id(0)

    @pl.when(step == 0)
    def _init():
        acc2_ref[...] = jnp.zeros_like(acc2_ref)
        ps1_ref[...] = jnp.zeros_like(ps1_ref)
        fs1_ref[...] = jnp.zeros_like(fs1_ref)

    @pl.when(step < _GRID_H)
    def _half2_probs():
        logits = jnp.dot(x_ref[...], wt_ref[...],
                         preferred_element_type=jnp.float32)
        logits = logits + bt_ref[...]
        m = jnp.max(logits, axis=-1, keepdims=True)
        e = jnp.exp(logits - m)
        s = jnp.sum(e, axis=-1, keepdims=True)
        probs = e / s
        probs_ref[...] = probs
        acc2_ref[...] += jnp.sum(probs, axis=0, keepdims=True)

    @pl.when(step == _GRID_H - 1)
    def _fin2():
        psum2_ref[...] = acc2_ref[...]

    @pl.when(step >= _GRID_H)
    def _half1_fused():
        logits_t = lax.dot_general(
            w_ref[...], x_ref[...],
            dimension_numbers=(((1,), (1,)), ((), ())),
            preferred_element_type=jnp.float32)
        logits_t = logits_t + b_ref[...]

        row = lax.broadcasted_iota(jnp.int32, (_E, _BT), 0)
        neg_inf = jnp.float32(-jnp.inf)
        big = jnp.int32(_E)

        work = logits_t
        vals = []
        idxs = []
        for _ in range(_K):
            mx = jnp.max(work, axis=0, keepdims=True)
            hit0 = work == mx
            rsel = jnp.min(jnp.where(hit0, row, big), axis=0, keepdims=True)
            vals.append(mx)
            idxs.append(rsel)
            work = jnp.where(row == rsel, neg_inf, work)

        v8 = jnp.concatenate(vals, axis=0)
        i8 = jnp.concatenate(idxs, axis=0)

        m0 = vals[0]
        e_t = jnp.exp(logits_t - m0)
        zinv = jnp.float32(1.0) / jnp.sum(e_t, axis=0, keepdims=True)
        probs_t = e_t * zinv
        ps1_ref[...] += jnp.sum(probs_t, axis=1, keepdims=True)

        sel = (work == neg_inf).astype(jnp.float32)
        fs1_ref[...] += jnp.sum(sel, axis=1, keepdims=True)

        p8 = jnp.exp(v8 - m0) * zinv
        s8 = jnp.sum(p8, axis=0, keepdims=True)
        out_p = p8 / (s8 + jnp.float32(1e-9))

        prob_ref[...] = out_p.T
        idx_ref[...] = i8.T

    @pl.when(step == _GRID - 1)
    def _fin1():
        psum1_ref[...] = ps1_ref[...]
        fsum1_ref[...] = fs1_ref[...]


def _tc_main(x, W, b):
    return pl.pallas_call(
        _tc_body,
        grid=(_GRID,),
        in_specs=[
            pl.BlockSpec((_E, _D), lambda i: (0, 0)),
            pl.BlockSpec((_D, _E), lambda i: (0, 0)),
            pl.BlockSpec((_BT, _D), lambda i: ((i + _GRID_H) % _GRID, 0)),
            pl.BlockSpec((_E, 1), lambda i: (0, 0)),
            pl.BlockSpec((1, _E), lambda i: (0, 0)),
        ],
        out_specs=[
            pl.BlockSpec((_BT, _E), lambda i: (jnp.minimum(i, _GRID_H - 1), 0)),
            pl.BlockSpec((1, _E), lambda i: (0, 0)),
            pl.BlockSpec((_BT, _K), lambda i: (jnp.maximum(i - _GRID_H, 0), 0)),
            pl.BlockSpec((_BT, _K), lambda i: (jnp.maximum(i - _GRID_H, 0), 0)),
            pl.BlockSpec((_E, 1), lambda i: (0, 0)),
            pl.BlockSpec((_E, 1), lambda i: (0, 0)),
        ],
        out_shape=[
            jax.ShapeDtypeStruct((_TH, _E), jnp.float32),
            jax.ShapeDtypeStruct((1, _E), jnp.float32),
            jax.ShapeDtypeStruct((_TH, _K), jnp.int32),
            jax.ShapeDtypeStruct((_TH, _K), jnp.float32),
            jax.ShapeDtypeStruct((_E, 1), jnp.float32),
            jax.ShapeDtypeStruct((_E, 1), jnp.float32),
        ],
        scratch_shapes=[
            pltpu.VMEM((1, _E), jnp.float32),
            pltpu.VMEM((_E, 1), jnp.float32),
            pltpu.VMEM((_E, 1), jnp.float32),
        ],
        compiler_params=pltpu.CompilerParams(
            dimension_semantics=("arbitrary",),
        ),
    )(W, W.T, x, b.reshape(_E, 1), b.reshape(1, _E))


def _merge_top16(ka, va, kb, vb):
    kbr = lax.rev(kb, (0,))
    vbr = lax.rev(vb, (0,))
    ga = ka >= kbr
    km = jnp.where(ga, ka, kbr)
    vm = jnp.where(ga, va, vbr)
    return plsc.sort_key_val(km, vm, descending=True)


def _sc_router_body(probs_hbm, idxf_hbm, probf_hbm, fcnt_hbm,
                    pv_ref, stg_p, stg_i, cnt_ref, sem):
    c = lax.axis_index("c")
    s = lax.axis_index("s")
    wid = s * 2 + c
    base_tok = wid * _TPW

    lane = lax.broadcasted_iota(jnp.int32, (16,), 0)
    first8 = lane < 8
    ones16 = jnp.ones((16,), jnp.float32)
    zeros16 = jnp.zeros((16,), jnp.float32)
    for j in range(4):
        cnt_ref[pl.ds(16 * j, 16)] = zeros16

    def chunk_body(ci, carry):
        tok0 = base_tok + ci * _CH
        pltpu.sync_copy(probs_hbm.at[pl.ds(tok0, _CH)], pv_ref)

        @plsc.parallel_loop(0, _CH, unroll=4)
        def tok_body(t):
            p0 = pv_ref[t, pl.ds(0, 16)]
            p1 = pv_ref[t, pl.ds(16, 16)]
            p2 = pv_ref[t, pl.ds(32, 16)]
            p3 = pv_ref[t, pl.ds(48, 16)]
            k0, v0 = plsc.sort_key_val(p0, lane, descending=True)
            k1, v1 = plsc.sort_key_val(p1, lane + 16, descending=True)
            k2, v2 = plsc.sort_key_val(p2, lane + 32, descending=True)
            k3, v3 = plsc.sort_key_val(p3, lane + 48, descending=True)
            ka, va = _merge_top16(k0, v0, k1, v1)
            kb, vb = _merge_top16(k2, v2, k3, v3)
            kf, vf = _merge_top16(ka, va, kb, vb)

            s8 = jnp.sum(jnp.where(first8, kf, zeros16))
            outp = kf / (s8 + jnp.float32(1e-9))

            off = pl.multiple_of(t * _K, 8)
            plsc.store_compressed(stg_p.at[pl.ds(off, 16)], outp, mask=first8)
            plsc.store_compressed(stg_i.at[pl.ds(off, 16)], vf, mask=first8)

        def cnt_body(g, carry2):
            iv = stg_i[pl.ds(g * 16, 16)]
            plsc.addupdate_scatter(cnt_ref, [iv], ones16)
            return carry2

        lax.fori_loop(0, _CH * _K // 16, cnt_body, 0)
        flat0 = pl.multiple_of(tok0 * _K, 8)
        pltpu.sync_copy(stg_p.at[pl.ds(0, _CH * _K)],
                        probf_hbm.at[pl.ds(flat0, _CH * _K)])
        pltpu.sync_copy(stg_i.at[pl.ds(0, _CH * _K)],
                        idxf_hbm.at[pl.ds(flat0, _CH * _K)])
        return carry

    lax.fori_loop(0, _NCHUNK, chunk_body, 0)
    pltpu.sync_copy(cnt_ref, fcnt_hbm.at[wid])


_sc_router = functools.partial(
    pl.kernel,
    mesh=plsc.VectorSubcoreMesh(core_axis_name="c", subcore_axis_name="s"),
    compiler_params=pltpu.CompilerParams(needs_layout_passes=False),
    out_type=[
        jax.ShapeDtypeStruct((_TH * _K,), jnp.int32),
        jax.ShapeDtypeStruct((_TH * _K,), jnp.float32),
        jax.ShapeDtypeStruct((_NW, _E), jnp.float32),
    ],
    scratch_types=[
        pltpu.VMEM((_CH, _E), jnp.float32),
        pltpu.VMEM((_CH * _K + 8,), jnp.float32),
        pltpu.VMEM((_CH * _K + 8,), jnp.int32),
        pltpu.VMEM((_E,), jnp.float32),
        pltpu.SemaphoreType.DMA,
    ],
)(_sc_router_body)


def _aux_body(p1_ref, f1_ref, p2_ref, fc2_ref, aux_ref):
    f2 = jnp.sum(fc2_ref[...], axis=0, keepdims=True)
    nn = ((), ())
    a = lax.dot_general(p1_ref[...], f1_ref[...], (((0,), (0,)), nn),
                        preferred_element_type=jnp.float32)
    bb = lax.dot_general(p2_ref[...], f2, (((1,), (1,)), nn),
                         preferred_element_type=jnp.float32)
    cc = lax.dot_general(f2, p1_ref[...], (((1,), (0,)), nn),
                         preferred_element_type=jnp.float32)
    dd = lax.dot_general(p2_ref[...], f1_ref[...], (((1,), (0,)), nn),
                         preferred_element_type=jnp.float32)
    scale = jnp.float32(float(_E) / (float(_T) * float(_T) * float(_K)))
    aux_ref[...] = scale * (a + bb + cc + dd)


def _tc_aux(psum1, fsum1, psum2, fcnt2):
    return pl.pallas_call(
        _aux_body,
        out_shape=jax.ShapeDtypeStruct((1, 1), jnp.float32),
    )(psum1, fsum1, psum2, fcnt2)


def kernel(x, W, b):
    probs2, psum2, idx1, prob1, psum1, fsum1 = _tc_main(x, W, b)
    idx2, prob2, fcnt2 = _sc_router(probs2)
    aux = _tc_aux(psum1, fsum1, psum2, fcnt2)
    idx = jnp.concatenate([idx1, idx2.reshape(_TH, _K)], axis=0)
    prob = jnp.concatenate([prob1, prob2.reshape(_TH, _K)], axis=0)
    return (idx, prob, aux[0, 0])
